# baseline (speedup 1.0000x reference)
import functools
import math

import jax
import jax.numpy as jnp
from jax import lax
from jax.experimental import pallas as pl
from jax.experimental.pallas import tpu as pltpu
from jax.experimental.pallas import tpu_sc as plsc

CB = 80
IB = 5
NODE_BLK = 1024


def _mm_bias_body(a_ref, w_ref, b_ref, o_ref):
    o_ref[...] = (
        jnp.dot(a_ref[...], w_ref[...], preferred_element_type=jnp.float32)
        + b_ref[...]
    )


def _mm_bias(a, w, b2, blk):
    rows, d_in = a.shape
    d_out = w.shape[1]
    return pl.pallas_call(
        _mm_bias_body,
        grid=(rows // blk,),
        in_specs=[
            pl.BlockSpec((blk, d_in), lambda i: (i, 0)),
            pl.BlockSpec((d_in, d_out), lambda i: (0, 0)),
            pl.BlockSpec((1, d_out), lambda i: (0, 0)),
        ],
        out_specs=pl.BlockSpec((blk, d_out), lambda i: (i, 0)),
        out_shape=jax.ShapeDtypeStruct((rows, d_out), jnp.float32),
    )(a, w, b2)


def _update_body(x_ref, agg_ref, cnt_ref, w1_ref, w2_ref, b_ref, g_ref,
                 bt_ref, o_ref):
    xb = x_ref[...]
    agg = agg_ref[0] + agg_ref[1]
    cnt8 = cnt_ref[0, 0] + cnt_ref[1, 0]
    nb = xb.shape[0]
    rows = lax.broadcasted_iota(jnp.int32, (nb, 8), 0) // 128
    sel = (rows == lax.broadcasted_iota(jnp.int32, (nb, 8), 1)).astype(
        jnp.float32)
    m = jnp.dot(sel, cnt8, preferred_element_type=jnp.float32)
    lanes = lax.broadcasted_iota(jnp.int32, (nb, 128), 0) % 128
    lmask = lanes == lax.broadcasted_iota(jnp.int32, (nb, 128), 1)
    c = jnp.sum(jnp.where(lmask, m, 0.0), axis=1, keepdims=True)
    mean = jnp.where(c > 0.0, agg / jnp.maximum(c, 1.0), 0.0)
    h = (
        jnp.dot(xb, w1_ref[...], preferred_element_type=jnp.float32)
        + jnp.dot(mean, w2_ref[...], preferred_element_type=jnp.float32)
        + b_ref[...]
    )
    mu = jnp.mean(h, axis=1, keepdims=True)
    d = h - mu
    var = jnp.mean(d * d, axis=1, keepdims=True)
    hn = d * lax.rsqrt(var + 1e-5) * g_ref[...] + bt_ref[...]
    o_ref[...] = jnp.maximum(hn + xb, 0.0)


@functools.lru_cache(maxsize=None)
def _make_sc_aggregate(nc, ns, k_chunks, n_pad, d):
    r = n_pad // ns
    nvec = d // 16
    mesh = plsc.VectorSubcoreMesh(core_axis_name="c", subcore_axis_name="s")

    @functools.partial(
        pl.kernel,
        out_type=(
            jax.ShapeDtypeStruct((nc, n_pad, d), jnp.float32),
            jax.ShapeDtypeStruct((nc, n_pad), jnp.float32),
        ),
        mesh=mesh,
        scratch_types=[
            pltpu.VMEM((2, 1, IB, CB), jnp.int32),
            pltpu.VMEM((2, 1, IB, CB), jnp.int32),
            pltpu.VMEM((CB, d), jnp.float32),
            pltpu.VMEM((CB, d), jnp.float32),
            pltpu.VMEM((CB, d), jnp.float32),
            pltpu.VMEM((CB, d), jnp.float32),
            pltpu.VMEM((CB,), jnp.float32),
            pltpu.VMEM((r,), jnp.float32),
            pltpu.VMEM_SHARED((n_pad, d), jnp.float32),
            pltpu.VMEM_SHARED((n_pad,), jnp.float32),
            pltpu.SemaphoreType.DMA,
            pltpu.SemaphoreType.DMA,
            pltpu.SemaphoreType.DMA,
            pltpu.SemaphoreType.DMA,
            pltpu.SemaphoreType.DMA,
            pltpu.SemaphoreType.DMA,
        ],
    )
    def sc_fn(t_hbm, xw1_hbm, src_hbm, dst_hbm, agg_out, cnt_out,
              src_v, dst_v, t_v0, t_v1, gx_v0, gx_v1,
              ones_v, czero_v, agg_sh, cnt_sh,
              sem_t0, sem_t1, sem_g0, sem_g1, sem_s0, sem_s1):
        c = lax.axis_index("c")
        s = lax.axis_index("s")
        wid = c * ns + s
        t_v = (t_v0, t_v1)
        gx_v = (gx_v0, gx_v1)
        sem_t = (sem_t0, sem_t1)
        sem_g = (sem_g0, sem_g1)
        sem_s = (sem_s0, sem_s1)
        ebase = wid * (k_chunks * CB)

        def init_row(i, carry):
            for cc in range(nvec):
                t_v0[i, pl.ds(cc * 16, 16)] = jnp.zeros((16,), jnp.float32)
            return carry

        lax.fori_loop(0, CB, init_row, 0)
        for cc in range(CB // 16):
            ones_v[pl.ds(cc * 16, 16)] = jnp.ones((16,), jnp.float32)

        def init_cz(i, carry):
            czero_v[pl.ds(i * 16, 16)] = jnp.zeros((16,), jnp.float32)
            return carry

        lax.fori_loop(0, r // 16, init_cz, 0)

        for kk in range(r // CB):
            pltpu.sync_copy(t_v0, agg_sh.at[pl.ds(s * r + kk * CB, CB)])
        pltpu.sync_copy(czero_v, cnt_sh.at[pl.ds(s * r, r)])
        plsc.subcore_barrier()

        def t_slice(j):
            return t_hbm.at[pl.ds(ebase + j * CB, CB)]

        def src_idx(j):
            return src_v.at[(j // IB) % 2, 0, j % IB]

        def dst_idx(j):
            return dst_v.at[(j // IB) % 2, 0, j % IB]

        def load_batch(j):
            b = j // IB

            @pl.when(j % IB == 0)
            def _():
                pltpu.sync_copy(src_hbm.at[wid, pl.ds(b, 1)],
                                src_v.at[b % 2])
                pltpu.sync_copy(dst_hbm.at[wid, pl.ds(b, 1)],
                                dst_v.at[b % 2])

        def wait_scatter(j, p):
            pltpu.make_async_copy(
                gx_v[p], agg_sh.at[dst_idx(j)], sem_s[p]).wait()
            pltpu.make_async_copy(
                ones_v, cnt_sh.at[dst_idx(j)], sem_s[p]).wait()

        def step(j, p):
            pltpu.make_async_copy(t_slice(j), t_v[p], sem_t[p]).wait()
            pltpu.make_async_copy(
                xw1_hbm.at[src_idx(j)], gx_v[p], sem_g[p]).wait()

            def row(i, carry2):
                for cc in range(nvec):
                    sl = pl.ds(cc * 16, 16)
                    gx_v[p][i, sl] = jnp.maximum(
                        t_v[p][i, sl] + gx_v[p][i, sl], 0.0)
                return carry2

            lax.fori_loop(0, CB, row, 0, unroll=2)

            pltpu.async_copy(gx_v[p], agg_sh.at[dst_idx(j)], sem_s[p],
                             add=True)
            pltpu.async_copy(ones_v, cnt_sh.at[dst_idx(j)], sem_s[p],
                             add=True)

            @pl.when(j + 2 < k_chunks)
            def _():
                pltpu.async_copy(t_slice(j + 2), t_v[p], sem_t[p])

            @pl.when(j >= 1)
            def _():
                wait_scatter(j - 1, 1 - p)

            @pl.when(j + 1 < k_chunks)
            def _():
                load_batch(j + 1)
                pltpu.async_copy(xw1_hbm.at[src_idx(j + 1)], gx_v[1 - p],
                                 sem_g[1 - p])

        load_batch(0)
        pltpu.async_copy(t_slice(0), t_v[0], sem_t[0])
        pltpu.async_copy(t_slice(1), t_v[1], sem_t[1])
        pltpu.async_copy(xw1_hbm.at[src_idx(0)], gx_v[0], sem_g[0])

        def pair(g, carry):
            j = 2 * g
            step(j, 0)
            step(j + 1, 1)
            return carry

        lax.fori_loop(0, (k_chunks - 1) // 2, pair, 0)
        step(k_chunks - 1, 0)
        wait_scatter(k_chunks - 1, 0)
        plsc.subcore_barrier()

        pltpu.sync_copy(agg_sh.at[pl.ds(s * r, r)],
                        agg_out.at[c, pl.ds(s * r, r)])
        pltpu.sync_copy(cnt_sh.at[pl.ds(s * r, r)],
                        cnt_out.at[c, pl.ds(s * r, r)])

    return sc_fn


def kernel(x, edge_index, edge_attr, W_msg, b_msg, W_upd, b_upd, ln_gamma,
           ln_beta):
    n, d = x.shape
    e = edge_index.shape[1]
    d_out = W_msg.shape[1]
    f32 = jnp.float32

    info = plsc.get_sparse_core_info()
    nc, ns = info.num_cores, info.num_subcores
    nw = nc * ns

    k_chunks = -(-e // (nw * CB))
    while k_chunks % 2 == 0 or k_chunks % IB != 0:
        k_chunks += 1
    e_pad = nw * k_chunks * CB
    align = math.lcm(NODE_BLK, ns * CB)
    n_pad = -(-(n + 1) // align) * align

    w1 = W_msg[:d]
    w2 = W_msg[d:]
    b2 = b_msg.reshape(1, d_out)
    zero_b = jnp.zeros((1, d_out), f32)

    x_pad = jnp.concatenate([x, jnp.zeros((n_pad - n, d), f32)], axis=0)
    xw1 = _mm_bias(x_pad, w1, zero_b, NODE_BLK)

    if e_pad > e:
        ea_pad = jnp.concatenate(
            [edge_attr, jnp.zeros((e_pad - e, d), f32)], axis=0)
        src = jnp.concatenate(
            [edge_index[0], jnp.zeros((e_pad - e,), jnp.int32)])
        dst = jnp.concatenate(
            [edge_index[1], jnp.full((e_pad - e,), n, jnp.int32)])
    else:
        ea_pad = edge_attr
        src = edge_index[0]
        dst = edge_index[1]
    t = _mm_bias(ea_pad, w2, b2, nw * CB)
    src3 = src.reshape(nw, k_chunks // IB, IB, CB)
    dst3 = dst.reshape(nw, k_chunks // IB, IB, CB)

    sc_fn = _make_sc_aggregate(nc, ns, k_chunks, n_pad, d_out)
    agg2, cnt2 = sc_fn(t, xw1, src3, dst3)
    cnt4 = cnt2.reshape(nc, n_pad // NODE_BLK, NODE_BLK // 128, 128)

    wu1 = W_upd[:d]
    wu2 = W_upd[d:]
    out = pl.pallas_call(
        _update_body,
        grid=(n_pad // NODE_BLK,),
        in_specs=[
            pl.BlockSpec((NODE_BLK, d), lambda i: (i, 0)),
            pl.BlockSpec((nc, NODE_BLK, d_out), lambda i: (0, i, 0)),
            pl.BlockSpec((nc, 1, NODE_BLK // 128, 128), lambda i: (0, i, 0, 0)),
            pl.BlockSpec((d, d_out), lambda i: (0, 0)),
            pl.BlockSpec((d_out, d_out), lambda i: (0, 0)),
            pl.BlockSpec((1, d_out), lambda i: (0, 0)),
            pl.BlockSpec((1, d_out), lambda i: (0, 0)),
            pl.BlockSpec((1, d_out), lambda i: (0, 0)),
        ],
        out_specs=pl.BlockSpec((NODE_BLK, d_out), lambda i: (i, 0)),
        out_shape=jax.ShapeDtypeStruct((n_pad, d_out), f32),
    )(x_pad, agg2, cnt4, wu1, wu2, b_upd.reshape(1, d_out),
      ln_gamma.reshape(1, d_out), ln_beta.reshape(1, d_out))

    return out[:n]


# speedup vs baseline: 1.1429x; 1.1429x over previous
import functools
import math

import jax
import jax.numpy as jnp
from jax import lax
from jax.experimental import pallas as pl
from jax.experimental.pallas import tpu as pltpu
from jax.experimental.pallas import tpu_sc as plsc

CB = 80
IB = 5
NODE_BLK = 1024


def _mm_bias_body(a_ref, w_ref, b_ref, o_ref):
    o_ref[...] = (
        jnp.dot(a_ref[...], w_ref[...], preferred_element_type=jnp.float32)
        + b_ref[...]
    )


def _mm_bias(a, w, b2, blk):
    rows, d_in = a.shape
    d_out = w.shape[1]
    return pl.pallas_call(
        _mm_bias_body,
        grid=(rows // blk,),
        in_specs=[
            pl.BlockSpec((blk, d_in), lambda i: (i, 0)),
            pl.BlockSpec((d_in, d_out), lambda i: (0, 0)),
            pl.BlockSpec((1, d_out), lambda i: (0, 0)),
        ],
        out_specs=pl.BlockSpec((blk, d_out), lambda i: (i, 0)),
        out_shape=jax.ShapeDtypeStruct((rows, d_out), jnp.float32),
    )(a, w, b2)


def _update_body(x_ref, agg_ref, cnt_ref, w1_ref, w2_ref, b_ref, g_ref,
                 bt_ref, o_ref):
    xb = x_ref[...]
    agg = agg_ref[0] + agg_ref[1]
    cnt8 = cnt_ref[0, 0] + cnt_ref[1, 0]
    nb = xb.shape[0]
    rows = lax.broadcasted_iota(jnp.int32, (nb, 8), 0) // 128
    sel = (rows == lax.broadcasted_iota(jnp.int32, (nb, 8), 1)).astype(
        jnp.float32)
    m = jnp.dot(sel, cnt8, preferred_element_type=jnp.float32)
    lanes = lax.broadcasted_iota(jnp.int32, (nb, 128), 0) % 128
    lmask = lanes == lax.broadcasted_iota(jnp.int32, (nb, 128), 1)
    c = jnp.sum(jnp.where(lmask, m, 0.0), axis=1, keepdims=True)
    mean = jnp.where(c > 0.0, agg / jnp.maximum(c, 1.0), 0.0)
    h = (
        jnp.dot(xb, w1_ref[...], preferred_element_type=jnp.float32)
        + jnp.dot(mean, w2_ref[...], preferred_element_type=jnp.float32)
        + b_ref[...]
    )
    mu = jnp.mean(h, axis=1, keepdims=True)
    d = h - mu
    var = jnp.mean(d * d, axis=1, keepdims=True)
    hn = d * lax.rsqrt(var + 1e-5) * g_ref[...] + bt_ref[...]
    o_ref[...] = jnp.maximum(hn + xb, 0.0)


@functools.lru_cache(maxsize=None)
def _make_sc_aggregate(nc, ns, k_chunks, n_pad, d):
    r = n_pad // ns
    nvec = d // 16
    mesh = plsc.VectorSubcoreMesh(core_axis_name="c", subcore_axis_name="s")

    @functools.partial(
        pl.kernel,
        out_type=(
            jax.ShapeDtypeStruct((nc, n_pad, d), jnp.float32),
            jax.ShapeDtypeStruct((nc, n_pad), jnp.float32),
        ),
        mesh=mesh,
        scratch_types=[
            pltpu.VMEM((2, 1, IB, CB), jnp.int32),
            pltpu.VMEM((2, 1, IB, CB), jnp.int32),
            pltpu.VMEM((CB, d), jnp.float32),
            pltpu.VMEM((CB, d), jnp.float32),
            pltpu.VMEM((CB, d), jnp.float32),
            pltpu.VMEM((CB, d), jnp.float32),
            pltpu.VMEM((CB,), jnp.float32),
            pltpu.VMEM((r,), jnp.float32),
            pltpu.VMEM_SHARED((n_pad, d), jnp.float32),
            pltpu.VMEM_SHARED((n_pad,), jnp.float32),
            pltpu.SemaphoreType.DMA,
            pltpu.SemaphoreType.DMA,
            pltpu.SemaphoreType.DMA,
            pltpu.SemaphoreType.DMA,
            pltpu.SemaphoreType.DMA,
            pltpu.SemaphoreType.DMA,
        ],
    )
    def sc_fn(t_hbm, xw1_hbm, src_hbm, dst_hbm, agg_out, cnt_out,
              src_v, dst_v, t_v0, t_v1, gx_v0, gx_v1,
              ones_v, czero_v, agg_sh, cnt_sh,
              sem_t0, sem_t1, sem_g0, sem_g1, sem_s0, sem_s1):
        c = lax.axis_index("c")
        s = lax.axis_index("s")
        wid = c * ns + s
        t_v = (t_v0, t_v1)
        gx_v = (gx_v0, gx_v1)
        sem_t = (sem_t0, sem_t1)
        sem_g = (sem_g0, sem_g1)
        sem_s = (sem_s0, sem_s1)
        ebase = wid * (k_chunks * CB)

        def init_row(i, carry):
            for cc in range(nvec):
                t_v0[i, pl.ds(cc * 16, 16)] = jnp.zeros((16,), jnp.float32)
            return carry

        lax.fori_loop(0, CB, init_row, 0)
        for cc in range(CB // 16):
            ones_v[pl.ds(cc * 16, 16)] = jnp.ones((16,), jnp.float32)

        def init_cz(i, carry):
            czero_v[pl.ds(i * 16, 16)] = jnp.zeros((16,), jnp.float32)
            return carry

        lax.fori_loop(0, r // 16, init_cz, 0)

        for kk in range(r // CB):
            pltpu.sync_copy(t_v0, agg_sh.at[pl.ds(s * r + kk * CB, CB)])
        pltpu.sync_copy(czero_v, cnt_sh.at[pl.ds(s * r, r)])
        plsc.subcore_barrier()

        def t_slice(j):
            return t_hbm.at[pl.ds(ebase + j * CB, CB)]

        def src_idx(j):
            return src_v.at[(j // IB) % 2, 0, j % IB]

        def dst_idx(j):
            return dst_v.at[(j // IB) % 2, 0, j % IB]

        def start(j, p):
            b = j // IB

            @pl.when(j % IB == 0)
            def _():
                pltpu.sync_copy(src_hbm.at[wid, pl.ds(b, 1)],
                                src_v.at[b % 2])
                pltpu.sync_copy(dst_hbm.at[wid, pl.ds(b, 1)],
                                dst_v.at[b % 2])

            pltpu.async_copy(t_slice(j), t_v[p], sem_t[p])
            pltpu.async_copy(xw1_hbm.at[src_idx(j)], gx_v[p], sem_g[p])

        def finish(j, p):
            pltpu.make_async_copy(t_slice(j), t_v[p], sem_t[p]).wait()
            pltpu.make_async_copy(
                xw1_hbm.at[src_idx(j)], gx_v[p], sem_g[p]).wait()

            def row(i, carry2):
                for cc in range(nvec):
                    sl = pl.ds(cc * 16, 16)
                    t_v[p][i, sl] = jnp.maximum(
                        t_v[p][i, sl] + gx_v[p][i, sl], 0.0)
                return carry2

            lax.fori_loop(0, CB, row, 0, unroll=4)

            pltpu.async_copy(t_v[p], agg_sh.at[dst_idx(j)], sem_s[p],
                             add=True)
            pltpu.async_copy(ones_v, cnt_sh.at[dst_idx(j)], sem_s[p],
                             add=True)
            pltpu.make_async_copy(t_v[p], agg_sh.at[dst_idx(j)],
                                  sem_s[p]).wait()
            pltpu.make_async_copy(ones_v, cnt_sh.at[dst_idx(j)],
                                  sem_s[p]).wait()

        start(0, 0)

        def pair(g, carry):
            j = 2 * g
            start(j + 1, 1)
            finish(j, 0)
            start(j + 2, 0)
            finish(j + 1, 1)
            return carry

        lax.fori_loop(0, (k_chunks - 1) // 2, pair, 0)
        finish(k_chunks - 1, 0)
        plsc.subcore_barrier()

        pltpu.sync_copy(agg_sh.at[pl.ds(s * r, r)],
                        agg_out.at[c, pl.ds(s * r, r)])
        pltpu.sync_copy(cnt_sh.at[pl.ds(s * r, r)],
                        cnt_out.at[c, pl.ds(s * r, r)])

    return sc_fn


def kernel(x, edge_index, edge_attr, W_msg, b_msg, W_upd, b_upd, ln_gamma,
           ln_beta):
    n, d = x.shape
    e = edge_index.shape[1]
    d_out = W_msg.shape[1]
    f32 = jnp.float32

    info = plsc.get_sparse_core_info()
    nc, ns = info.num_cores, info.num_subcores
    nw = nc * ns

    k_chunks = -(-e // (nw * CB))
    while k_chunks % 2 == 0 or k_chunks % IB != 0:
        k_chunks += 1
    e_pad = nw * k_chunks * CB
    align = math.lcm(NODE_BLK, ns * CB)
    n_pad = -(-(n + 1) // align) * align

    w1 = W_msg[:d]
    w2 = W_msg[d:]
    b2 = b_msg.reshape(1, d_out)
    zero_b = jnp.zeros((1, d_out), f32)

    x_pad = jnp.concatenate([x, jnp.zeros((n_pad - n, d), f32)], axis=0)
    xw1 = _mm_bias(x_pad, w1, zero_b, NODE_BLK)

    if e_pad > e:
        ea_pad = jnp.concatenate(
            [edge_attr, jnp.zeros((e_pad - e, d), f32)], axis=0)
        src = jnp.concatenate(
            [edge_index[0], jnp.zeros((e_pad - e,), jnp.int32)])
        dst = jnp.concatenate(
            [edge_index[1], jnp.full((e_pad - e,), n, jnp.int32)])
    else:
        ea_pad = edge_attr
        src = edge_index[0]
        dst = edge_index[1]
    t = _mm_bias(ea_pad, w2, b2, nw * CB)
    src3 = src.reshape(nw, k_chunks // IB, IB, CB)
    dst3 = dst.reshape(nw, k_chunks // IB, IB, CB)

    sc_fn = _make_sc_aggregate(nc, ns, k_chunks, n_pad, d_out)
    agg2, cnt2 = sc_fn(t, xw1, src3, dst3)
    cnt4 = cnt2.reshape(nc, n_pad // NODE_BLK, NODE_BLK // 128, 128)

    wu1 = W_upd[:d]
    wu2 = W_upd[d:]
    out = pl.pallas_call(
        _update_body,
        grid=(n_pad // NODE_BLK,),
        in_specs=[
            pl.BlockSpec((NODE_BLK, d), lambda i: (i, 0)),
            pl.BlockSpec((nc, NODE_BLK, d_out), lambda i: (0, i, 0)),
            pl.BlockSpec((nc, 1, NODE_BLK // 128, 128), lambda i: (0, i, 0, 0)),
            pl.BlockSpec((d, d_out), lambda i: (0, 0)),
            pl.BlockSpec((d_out, d_out), lambda i: (0, 0)),
            pl.BlockSpec((1, d_out), lambda i: (0, 0)),
            pl.BlockSpec((1, d_out), lambda i: (0, 0)),
            pl.BlockSpec((1, d_out), lambda i: (0, 0)),
        ],
        out_specs=pl.BlockSpec((NODE_BLK, d_out), lambda i: (i, 0)),
        out_shape=jax.ShapeDtypeStruct((n_pad, d_out), f32),
    )(x_pad, agg2, cnt4, wu1, wu2, b_upd.reshape(1, d_out),
      ln_gamma.reshape(1, d_out), ln_beta.reshape(1, d_out))

    return out[:n]


# speedup vs baseline: 1.8861x; 1.6503x over previous
import functools
import math

import jax
import jax.numpy as jnp
from jax import lax
from jax.experimental import pallas as pl
from jax.experimental.pallas import tpu as pltpu
from jax.experimental.pallas import tpu_sc as plsc

CB = 80
IB = 5
NODE_BLK = 1024


def _mm_bias_body(a_ref, w_ref, b_ref, o_ref):
    o_ref[...] = (
        jnp.dot(a_ref[...], w_ref[...], preferred_element_type=jnp.float32)
        + b_ref[...]
    )


def _mm_bias(a, w, b2, blk):
    rows, d_in = a.shape
    d_out = w.shape[1]
    return pl.pallas_call(
        _mm_bias_body,
        grid=(rows // blk,),
        in_specs=[
            pl.BlockSpec((blk, d_in), lambda i: (i, 0)),
            pl.BlockSpec((d_in, d_out), lambda i: (0, 0)),
            pl.BlockSpec((1, d_out), lambda i: (0, 0)),
        ],
        out_specs=pl.BlockSpec((blk, d_out), lambda i: (i, 0)),
        out_shape=jax.ShapeDtypeStruct((rows, d_out), jnp.float32),
    )(a, w, b2)


def _update_body(x_ref, agg_ref, cnt_ref, w1_ref, w2_ref, b_ref, g_ref,
                 bt_ref, o_ref):
    xb = x_ref[...]
    agg = agg_ref[0] + agg_ref[1]
    cnt8 = cnt_ref[0, 0] + cnt_ref[1, 0]
    nb = xb.shape[0]
    rows = lax.broadcasted_iota(jnp.int32, (nb, 8), 0) // 128
    sel = (rows == lax.broadcasted_iota(jnp.int32, (nb, 8), 1)).astype(
        jnp.float32)
    m = jnp.dot(sel, cnt8, preferred_element_type=jnp.float32)
    lanes = lax.broadcasted_iota(jnp.int32, (nb, 128), 0) % 128
    lmask = lanes == lax.broadcasted_iota(jnp.int32, (nb, 128), 1)
    c = jnp.sum(jnp.where(lmask, m, 0.0), axis=1, keepdims=True)
    mean = jnp.where(c > 0.0, agg / jnp.maximum(c, 1.0), 0.0)
    h = (
        jnp.dot(xb, w1_ref[...], preferred_element_type=jnp.float32)
        + jnp.dot(mean, w2_ref[...], preferred_element_type=jnp.float32)
        + b_ref[...]
    )
    mu = jnp.mean(h, axis=1, keepdims=True)
    d = h - mu
    var = jnp.mean(d * d, axis=1, keepdims=True)
    hn = d * lax.rsqrt(var + 1e-5) * g_ref[...] + bt_ref[...]
    o_ref[...] = jnp.maximum(hn + xb, 0.0)


@functools.lru_cache(maxsize=None)
def _make_sc_aggregate(nc, ns, k_chunks, n_pad, d):
    r = n_pad // ns
    nvec = d // 16
    mesh = plsc.VectorSubcoreMesh(core_axis_name="c", subcore_axis_name="s")

    @functools.partial(
        pl.kernel,
        out_type=(
            jax.ShapeDtypeStruct((nc, n_pad, d), jnp.float32),
            jax.ShapeDtypeStruct((nc, n_pad), jnp.float32),
        ),
        mesh=mesh,
        scratch_types=[
            pltpu.VMEM((2, 1, IB, CB), jnp.int32),
            pltpu.VMEM((2, 1, IB, CB), jnp.int32),
            pltpu.VMEM((CB, d), jnp.float32),
            pltpu.VMEM((CB, d), jnp.float32),
            pltpu.VMEM((CB, d), jnp.float32),
            pltpu.VMEM((CB, d), jnp.float32),
            pltpu.VMEM((CB,), jnp.float32),
            pltpu.VMEM((r,), jnp.float32),
            pltpu.VMEM_SHARED((n_pad, d), jnp.float32),
            pltpu.VMEM_SHARED((n_pad,), jnp.float32),
            pltpu.SemaphoreType.DMA,
            pltpu.SemaphoreType.DMA,
            pltpu.SemaphoreType.DMA,
            pltpu.SemaphoreType.DMA,
            pltpu.SemaphoreType.DMA,
            pltpu.SemaphoreType.DMA,
        ],
    )
    def sc_fn(t_hbm, xw1_hbm, src_hbm, dst_hbm, agg_out, cnt_out,
              src_v, dst_v, t_v0, t_v1, gx_v0, gx_v1,
              ones_v, czero_v, agg_sh, cnt_sh,
              sem_t0, sem_t1, sem_g0, sem_g1, sem_s0, sem_s1):
        c = lax.axis_index("c")
        s = lax.axis_index("s")
        wid = c * ns + s
        t_v = (t_v0, t_v1)
        gx_v = (gx_v0, gx_v1)
        sem_t = (sem_t0, sem_t1)
        sem_g = (sem_g0, sem_g1)
        sem_s = (sem_s0, sem_s1)
        ebase = wid * (k_chunks * CB)

        def init_row(i, carry):
            for cc in range(nvec):
                t_v0[i, pl.ds(cc * 16, 16)] = jnp.zeros((16,), jnp.float32)
            return carry

        lax.fori_loop(0, CB, init_row, 0)
        for cc in range(CB // 16):
            ones_v[pl.ds(cc * 16, 16)] = jnp.ones((16,), jnp.float32)

        def init_cz(i, carry):
            czero_v[pl.ds(i * 16, 16)] = jnp.zeros((16,), jnp.float32)
            return carry

        lax.fori_loop(0, r // 16, init_cz, 0)

        for kk in range(r // CB):
            pltpu.sync_copy(t_v0, agg_sh.at[pl.ds(s * r + kk * CB, CB)])
        pltpu.sync_copy(czero_v, cnt_sh.at[pl.ds(s * r, r)])
        plsc.subcore_barrier()

        def t_slice(j):
            return t_hbm.at[pl.ds(ebase + j * CB, CB)]

        def src_idx(j):
            return src_v.at[(j // IB) % 2, 0, j % IB]

        def dst_idx(j):
            return dst_v.at[(j // IB) % 2, 0, j % IB]

        def start(j, p):
            b = j // IB

            @pl.when(j % IB == 0)
            def _():
                pltpu.sync_copy(src_hbm.at[wid, pl.ds(b, 1)],
                                src_v.at[b % 2])
                pltpu.sync_copy(dst_hbm.at[wid, pl.ds(b, 1)],
                                dst_v.at[b % 2])

            pltpu.async_copy(t_slice(j), t_v[p], sem_t[p])
            pltpu.async_copy(xw1_hbm.at[src_idx(j)], gx_v[p], sem_g[p])

        def finish(j, p):
            pltpu.make_async_copy(t_slice(j), t_v[p], sem_t[p]).wait()
            pltpu.make_async_copy(
                xw1_hbm.at[src_idx(j)], gx_v[p], sem_g[p]).wait()

            @plsc.parallel_loop(0, CB, unroll=2)
            def row(i):
                for cc in range(nvec):
                    sl = pl.ds(cc * 16, 16)
                    t_v[p][i, sl] = jnp.maximum(
                        t_v[p][i, sl] + gx_v[p][i, sl], 0.0)

            pltpu.async_copy(t_v[p], agg_sh.at[dst_idx(j)], sem_s[p],
                             add=True)
            pltpu.async_copy(ones_v, cnt_sh.at[dst_idx(j)], sem_s[p],
                             add=True)
            pltpu.make_async_copy(t_v[p], agg_sh.at[dst_idx(j)],
                                  sem_s[p]).wait()
            pltpu.make_async_copy(ones_v, cnt_sh.at[dst_idx(j)],
                                  sem_s[p]).wait()

        start(0, 0)

        def pair(g, carry):
            j = 2 * g
            start(j + 1, 1)
            finish(j, 0)
            start(j + 2, 0)
            finish(j + 1, 1)
            return carry

        lax.fori_loop(0, (k_chunks - 1) // 2, pair, 0)
        finish(k_chunks - 1, 0)
        plsc.subcore_barrier()

        pltpu.sync_copy(agg_sh.at[pl.ds(s * r, r)],
                        agg_out.at[c, pl.ds(s * r, r)])
        pltpu.sync_copy(cnt_sh.at[pl.ds(s * r, r)],
                        cnt_out.at[c, pl.ds(s * r, r)])

    return sc_fn


def kernel(x, edge_index, edge_attr, W_msg, b_msg, W_upd, b_upd, ln_gamma,
           ln_beta):
    n, d = x.shape
    e = edge_index.shape[1]
    d_out = W_msg.shape[1]
    f32 = jnp.float32

    info = plsc.get_sparse_core_info()
    nc, ns = info.num_cores, info.num_subcores
    nw = nc * ns

    k_chunks = -(-e // (nw * CB))
    while k_chunks % 2 == 0 or k_chunks % IB != 0:
        k_chunks += 1
    e_pad = nw * k_chunks * CB
    align = math.lcm(NODE_BLK, ns * CB)
    n_pad = -(-(n + 1) // align) * align

    w1 = W_msg[:d]
    w2 = W_msg[d:]
    b2 = b_msg.reshape(1, d_out)
    zero_b = jnp.zeros((1, d_out), f32)

    x_pad = jnp.concatenate([x, jnp.zeros((n_pad - n, d), f32)], axis=0)
    xw1 = _mm_bias(x_pad, w1, zero_b, NODE_BLK)

    if e_pad > e:
        ea_pad = jnp.concatenate(
            [edge_attr, jnp.zeros((e_pad - e, d), f32)], axis=0)
        src = jnp.concatenate(
            [edge_index[0], jnp.zeros((e_pad - e,), jnp.int32)])
        dst = jnp.concatenate(
            [edge_index[1], jnp.full((e_pad - e,), n, jnp.int32)])
    else:
        ea_pad = edge_attr
        src = edge_index[0]
        dst = edge_index[1]
    t = _mm_bias(ea_pad, w2, b2, nw * CB)
    src3 = src.reshape(nw, k_chunks // IB, IB, CB)
    dst3 = dst.reshape(nw, k_chunks // IB, IB, CB)

    sc_fn = _make_sc_aggregate(nc, ns, k_chunks, n_pad, d_out)
    agg2, cnt2 = sc_fn(t, xw1, src3, dst3)
    cnt4 = cnt2.reshape(nc, n_pad // NODE_BLK, NODE_BLK // 128, 128)

    wu1 = W_upd[:d]
    wu2 = W_upd[d:]
    out = pl.pallas_call(
        _update_body,
        grid=(n_pad // NODE_BLK,),
        in_specs=[
            pl.BlockSpec((NODE_BLK, d), lambda i: (i, 0)),
            pl.BlockSpec((nc, NODE_BLK, d_out), lambda i: (0, i, 0)),
            pl.BlockSpec((nc, 1, NODE_BLK // 128, 128), lambda i: (0, i, 0, 0)),
            pl.BlockSpec((d, d_out), lambda i: (0, 0)),
            pl.BlockSpec((d_out, d_out), lambda i: (0, 0)),
            pl.BlockSpec((1, d_out), lambda i: (0, 0)),
            pl.BlockSpec((1, d_out), lambda i: (0, 0)),
            pl.BlockSpec((1, d_out), lambda i: (0, 0)),
        ],
        out_specs=pl.BlockSpec((NODE_BLK, d_out), lambda i: (i, 0)),
        out_shape=jax.ShapeDtypeStruct((n_pad, d_out), f32),
    )(x_pad, agg2, cnt4, wu1, wu2, b_upd.reshape(1, d_out),
      ln_gamma.reshape(1, d_out), ln_beta.reshape(1, d_out))

    return out[:n]


# speedup vs baseline: 2.2592x; 1.1978x over previous
import functools
import math

import jax
import jax.numpy as jnp
import numpy as np
from jax import lax
from jax.experimental import pallas as pl
from jax.experimental.pallas import tpu as pltpu
from jax.experimental.pallas import tpu_sc as plsc

CB = 80
IB = 5
NODE_BLK = 1024


def _mm_bias_body(a_ref, w_ref, b_ref, o_ref):
    o_ref[...] = (
        jnp.dot(a_ref[...], w_ref[...], preferred_element_type=jnp.float32)
        + b_ref[...]
    )


def _mm_bias_pack(a, w, b2, blk, cb, row_off, n_rows):
    d_in = a.shape[1]
    d_out = w.shape[1]
    off_b = row_off // blk

    def body(a_ref, w_ref, b_ref, o_ref):
        h = (
            jnp.dot(a_ref[...], w_ref[...],
                    preferred_element_type=jnp.float32)
            + b_ref[...]
        )
        h3 = h.reshape(blk // cb, cb, d_out)
        lo = h3[:, :cb // 2, :].reshape(blk // 2, d_out)
        hi = h3[:, cb // 2:, :].reshape(blk // 2, d_out)
        lo = lo.astype(jnp.bfloat16).astype(jnp.float32)
        hi = hi.astype(jnp.bfloat16).astype(jnp.float32)
        lo_b = lax.shift_right_logical(
            lax.bitcast_convert_type(lo, jnp.int32), 16)
        hi_b = lax.bitcast_convert_type(hi, jnp.int32) & jnp.int32(-65536)
        o_ref[...] = hi_b | lo_b

    return pl.pallas_call(
        body,
        grid=(n_rows // blk,),
        in_specs=[
            pl.BlockSpec((blk, d_in), lambda i: (i + off_b, 0)),
            pl.BlockSpec((d_in, d_out), lambda i: (0, 0)),
            pl.BlockSpec((1, d_out), lambda i: (0, 0)),
        ],
        out_specs=pl.BlockSpec((blk // 2, d_out), lambda i: (i, 0)),
        out_shape=jax.ShapeDtypeStruct((n_rows // 2, d_out), jnp.int32),
    )(a, w, b2)


def _mm_bias(a, w, b2, blk):
    rows, d_in = a.shape
    d_out = w.shape[1]
    return pl.pallas_call(
        _mm_bias_body,
        grid=(rows // blk,),
        in_specs=[
            pl.BlockSpec((blk, d_in), lambda i: (i, 0)),
            pl.BlockSpec((d_in, d_out), lambda i: (0, 0)),
            pl.BlockSpec((1, d_out), lambda i: (0, 0)),
        ],
        out_specs=pl.BlockSpec((blk, d_out), lambda i: (i, 0)),
        out_shape=jax.ShapeDtypeStruct((rows, d_out), jnp.float32),
    )(a, w, b2)


def _make_update_body(n_parts):
    def body(x_ref, *refs):
        agg_refs = refs[:n_parts]
        cnt_refs = refs[n_parts:2 * n_parts]
        w1_ref, w2_ref, b_ref, g_ref, bt_ref, o_ref = refs[2 * n_parts:]
        xb = x_ref[...]
        agg = agg_refs[0][0] + agg_refs[0][1]
        cnt8 = cnt_refs[0][0, 0] + cnt_refs[0][1, 0]
        for pi in range(1, n_parts):
            agg = agg + agg_refs[pi][0] + agg_refs[pi][1]
            cnt8 = cnt8 + cnt_refs[pi][0, 0] + cnt_refs[pi][1, 0]
        nb = xb.shape[0]
        rows = lax.broadcasted_iota(jnp.int32, (nb, 8), 0) // 128
        sel = (rows == lax.broadcasted_iota(jnp.int32, (nb, 8), 1)).astype(
            jnp.float32)
        m = jnp.dot(sel, cnt8, preferred_element_type=jnp.float32)
        lanes = lax.broadcasted_iota(jnp.int32, (nb, 128), 0) % 128
        lmask = lanes == lax.broadcasted_iota(jnp.int32, (nb, 128), 1)
        c = jnp.sum(jnp.where(lmask, m, 0.0), axis=1, keepdims=True)
        mean = jnp.where(c > 0.0, agg / jnp.maximum(c, 1.0), 0.0)
        h = (
            jnp.dot(xb, w1_ref[...], preferred_element_type=jnp.float32)
            + jnp.dot(mean, w2_ref[...], preferred_element_type=jnp.float32)
            + b_ref[...]
        )
        mu = jnp.mean(h, axis=1, keepdims=True)
        d = h - mu
        var = jnp.mean(d * d, axis=1, keepdims=True)
        hn = d * lax.rsqrt(var + 1e-5) * g_ref[...] + bt_ref[...]
        o_ref[...] = jnp.maximum(hn + xb, 0.0)
    return body


@functools.lru_cache(maxsize=None)
def _make_sc_aggregate(nc, ns, k_chunks, n_pad, d, cb):
    r = n_pad // ns
    nvec = d // 16
    ones_n = -(-cb // 16) * 16
    mesh = plsc.VectorSubcoreMesh(core_axis_name="c", subcore_axis_name="s")

    @functools.partial(
        pl.kernel,
        out_type=(
            jax.ShapeDtypeStruct((nc, n_pad, d), jnp.float32),
            jax.ShapeDtypeStruct((nc, n_pad), jnp.float32),
        ),
        mesh=mesh,
        compiler_params=pltpu.CompilerParams(needs_layout_passes=False),
        scratch_types=[
            pltpu.VMEM((2, 1, IB, cb), jnp.int32),
            pltpu.VMEM((2, 1, IB, cb), jnp.int32),
            pltpu.VMEM((cb // 2, d), jnp.int32),
            pltpu.VMEM((cb // 2, d), jnp.int32),
            pltpu.VMEM((cb, d), jnp.float32),
            pltpu.VMEM((cb, d), jnp.float32),
            pltpu.VMEM((ones_n,), jnp.float32),
            pltpu.VMEM((r,), jnp.float32),
            pltpu.VMEM_SHARED((n_pad, d), jnp.float32),
            pltpu.VMEM_SHARED((n_pad,), jnp.float32),
            pltpu.SemaphoreType.DMA,
            pltpu.SemaphoreType.DMA,
            pltpu.SemaphoreType.DMA,
            pltpu.SemaphoreType.DMA,
            pltpu.SemaphoreType.DMA,
            pltpu.SemaphoreType.DMA,
        ],
    )
    def sc_fn(t_hbm, xw1_hbm, src_hbm, dst_hbm, agg_out, cnt_out,
              src_v, dst_v, t_v0, t_v1, gx_v0, gx_v1,
              ones_v, czero_v, agg_sh, cnt_sh,
              sem_t0, sem_t1, sem_g0, sem_g1, sem_s0, sem_s1):
        c = lax.axis_index("c")
        s = lax.axis_index("s")
        wid = c * ns + s
        t_v = (t_v0, t_v1)
        gx_v = (gx_v0, gx_v1)
        sem_t = (sem_t0, sem_t1)
        sem_g = (sem_g0, sem_g1)
        sem_s = (sem_s0, sem_s1)

        def init_row(i, carry):
            for cc in range(nvec):
                gx_v0[i, pl.ds(cc * 16, 16)] = jnp.zeros((16,), jnp.float32)
            return carry

        lax.fori_loop(0, cb, init_row, 0)
        for cc in range(ones_n // 16):
            ones_v[pl.ds(cc * 16, 16)] = jnp.ones((16,), jnp.float32)

        def init_cz(i, carry):
            czero_v[pl.ds(i * 16, 16)] = jnp.zeros((16,), jnp.float32)
            return carry

        lax.fori_loop(0, r // 16, init_cz, 0)

        for kk in range(r // cb):
            pltpu.sync_copy(gx_v0, agg_sh.at[pl.ds(s * r + kk * cb, cb)])
        rem = r % cb
        if rem:
            pltpu.sync_copy(gx_v0.at[pl.ds(0, rem)],
                            agg_sh.at[pl.ds(s * r + (r // cb) * cb, rem)])
        pltpu.sync_copy(czero_v, cnt_sh.at[pl.ds(s * r, r)])
        plsc.subcore_barrier()

        def t_slice(j):
            return t_hbm.at[pl.ds(wid * (k_chunks * cb // 2) + j * (cb // 2),
                                  cb // 2)]

        def src_idx(j):
            return src_v.at[(j // IB) % 2, 0, j % IB]

        def dst_idx(j):
            return dst_v.at[(j // IB) % 2, 0, j % IB]

        def start(j, p, guard=False):
            def issue():
                b = j // IB

                @pl.when(j % IB == 0)
                def _():
                    pltpu.sync_copy(src_hbm.at[wid, pl.ds(b, 1)],
                                    src_v.at[b % 2])
                    pltpu.sync_copy(dst_hbm.at[wid, pl.ds(b, 1)],
                                    dst_v.at[b % 2])

                pltpu.async_copy(t_slice(j), t_v[p], sem_t[p])
                pltpu.async_copy(xw1_hbm.at[src_idx(j)], gx_v[p], sem_g[p])

            if guard:
                pl.when(j < k_chunks)(issue)
            else:
                issue()

        def finish(j, p):
            pltpu.make_async_copy(t_slice(j), t_v[p], sem_t[p]).wait()
            pltpu.make_async_copy(
                xw1_hbm.at[src_idx(j)], gx_v[p], sem_g[p]).wait()

            @plsc.parallel_loop(0, cb // 2, unroll=2)
            def rowpair(i2):
                for cc in range(nvec):
                    sl = pl.ds(cc * 16, 16)
                    w = t_v[p][i2, sl]
                    lo = plsc.bitcast(w << 16, jnp.float32)
                    hi = plsc.bitcast(w & jnp.int32(-65536), jnp.float32)
                    gx_v[p][i2, sl] = jnp.maximum(
                        gx_v[p][i2, sl] + lo, 0.0)
                    gx_v[p][i2 + cb // 2, sl] = jnp.maximum(
                        gx_v[p][i2 + cb // 2, sl] + hi, 0.0)

            pltpu.async_copy(gx_v[p], agg_sh.at[dst_idx(j)], sem_s[p],
                             add=True)
            pltpu.async_copy(ones_v.at[pl.ds(0, cb)], cnt_sh.at[dst_idx(j)],
                             sem_s[p], add=True)
            pltpu.make_async_copy(gx_v[p], agg_sh.at[dst_idx(j)],
                                  sem_s[p]).wait()
            pltpu.make_async_copy(ones_v.at[pl.ds(0, cb)],
                                  cnt_sh.at[dst_idx(j)], sem_s[p]).wait()

        start(0, 0)

        def pair(g, carry):
            j = 2 * g
            start(j + 1, 1)
            finish(j, 0)
            start(j + 2, 0, guard=(k_chunks % 2 == 0))
            finish(j + 1, 1)
            return carry

        lax.fori_loop(0, k_chunks // 2, pair, 0)
        if k_chunks % 2:
            finish(k_chunks - 1, 0)
        plsc.subcore_barrier()

        pltpu.sync_copy(agg_sh.at[pl.ds(s * r, r)],
                        agg_out.at[c, pl.ds(s * r, r)])
        pltpu.sync_copy(cnt_sh.at[pl.ds(s * r, r)],
                        cnt_out.at[c, pl.ds(s * r, r)])

    return sc_fn


def kernel(x, edge_index, edge_attr, W_msg, b_msg, W_upd, b_upd, ln_gamma,
           ln_beta):
    n, d = x.shape
    e = edge_index.shape[1]
    d_out = W_msg.shape[1]
    f32 = jnp.float32

    info = plsc.get_sparse_core_info()
    nc, ns = info.num_cores, info.num_subcores
    nw = nc * ns

    cb = CB
    k_total = e // (nw * cb)
    if e % (nw * cb) == 0 and k_total >= 4 * IB:
        k1 = (k_total // 2) // IB * IB
        ks = [k1, k_total - k1]
        e_pad = e
    else:
        k_chunks = -(-e // (nw * cb))
        while k_chunks % 2 == 0 or k_chunks % IB != 0:
            k_chunks += 1
        ks = [k_chunks]
        e_pad = nw * k_chunks * cb
    n_parts = len(ks)

    align = math.lcm(NODE_BLK, 16 * ns)
    n_pad = -(-(n + 1) // align) * align

    w1 = W_msg[:d]
    w2 = W_msg[d:]
    b2 = b_msg.reshape(1, d_out)
    zero_b = jnp.zeros((1, d_out), f32)

    x_pad = jnp.concatenate([x, jnp.zeros((n_pad - n, d), f32)], axis=0)
    xw1 = _mm_bias(x_pad, w1, zero_b, NODE_BLK)

    if e_pad > e:
        ea_pad = jnp.concatenate(
            [edge_attr, jnp.zeros((e_pad - e, d), f32)], axis=0)
        src = jnp.concatenate(
            [edge_index[0], jnp.zeros((e_pad - e,), jnp.int32)])
        dst = jnp.concatenate(
            [edge_index[1], jnp.full((e_pad - e,), n, jnp.int32)])
    else:
        ea_pad = edge_attr
        src = edge_index[0]
        dst = edge_index[1]

    aggs, cnts = [], []
    off = 0
    for pi in range(n_parts):
        k_i = ks[pi]
        e_part = nw * k_i * cb
        sc_fn = _make_sc_aggregate(nc, ns, k_i, n_pad, d_out, cb)
        t_p = _mm_bias_pack(ea_pad, w2, b2, nw * cb, cb, off, e_part)
        src3 = lax.dynamic_slice_in_dim(src, off, e_part).reshape(
            nw, k_i // IB, IB, cb)
        dst3 = lax.dynamic_slice_in_dim(dst, off, e_part).reshape(
            nw, k_i // IB, IB, cb)
        agg2, cnt2 = sc_fn(t_p, xw1, src3, dst3)
        aggs.append(agg2)
        cnts.append(cnt2.reshape(nc, n_pad // NODE_BLK, NODE_BLK // 128, 128))
        off += e_part

    wu1 = W_upd[:d]
    wu2 = W_upd[d:]
    in_specs = [pl.BlockSpec((NODE_BLK, d), lambda i: (i, 0))]
    in_specs += [pl.BlockSpec((nc, NODE_BLK, d_out), lambda i: (0, i, 0))
                 for _ in range(n_parts)]
    in_specs += [pl.BlockSpec((nc, 1, NODE_BLK // 128, 128),
                              lambda i: (0, i, 0, 0)) for _ in range(n_parts)]
    in_specs += [
        pl.BlockSpec((d, d_out), lambda i: (0, 0)),
        pl.BlockSpec((d_out, d_out), lambda i: (0, 0)),
        pl.BlockSpec((1, d_out), lambda i: (0, 0)),
        pl.BlockSpec((1, d_out), lambda i: (0, 0)),
        pl.BlockSpec((1, d_out), lambda i: (0, 0)),
    ]
    out = pl.pallas_call(
        _make_update_body(n_parts),
        grid=(n_pad // NODE_BLK,),
        in_specs=in_specs,
        out_specs=pl.BlockSpec((NODE_BLK, d_out), lambda i: (i, 0)),
        out_shape=jax.ShapeDtypeStruct((n_pad, d_out), f32),
    )(x_pad, *aggs, *cnts, wu1, wu2, b_upd.reshape(1, d_out),
      ln_gamma.reshape(1, d_out), ln_beta.reshape(1, d_out))

    return out[:n]
